# baseline (speedup 1.0000x reference)
import functools

import jax
import jax.numpy as jnp
from jax import lax
from jax.experimental import pallas as pl
from jax.experimental.pallas import tpu as pltpu
from jax.experimental.pallas import tpu_sc as plsc

NUM_ENTITIES = 1000000
NUM_RELATIONS = 1000
EMBED_DIM = 64
BATCH = 16384

NUM_CORES = 2
NUM_SUBCORES = 16
NUM_WORKERS = NUM_CORES * NUM_SUBCORES
B_PER_W = BATCH // NUM_WORKERS

_W = 32768
_Q = _W // 4
_LW = _W.bit_length() - 1

_NBLK = (NUM_ENTITIES + _W - 1) // _W
_NB_LO = 16
_S = _NB_LO * _W
_ROWS_LO = _NB_LO * _Q


def _tr_body(x_ref, o_ref):
    x = x_ref[...].astype(jnp.bfloat16)
    y = jnp.transpose(x)
    u = jax.lax.bitcast_convert_type(y, jnp.uint16).astype(jnp.int32)
    a, b, c, d = (u[0:_Q], u[_Q:2 * _Q], u[2 * _Q:3 * _Q], u[3 * _Q:4 * _Q])
    p1 = a | (c << 16)
    p2 = b | (d << 16)
    o_ref[...] = jnp.concatenate([p1, p2], axis=1)


def _transpose_pack(et, nblk, blk0):
    return pl.pallas_call(
        _tr_body,
        grid=(nblk,),
        in_specs=[pl.BlockSpec((64, _W), lambda i: (0, i + blk0))],
        out_specs=pl.BlockSpec((_Q, 128), lambda i: (i, 0)),
        out_shape=jax.ShapeDtypeStruct((nblk * _Q, 128), jnp.int32),
    )(et)


def _sc_gather(tabs_and_idx, n_out):
    mesh = plsc.VectorSubcoreMesh(
        core_axis_name="c", subcore_axis_name="s",
        num_cores=NUM_CORES, num_subcores=NUM_SUBCORES)
    out_ty = tuple(jax.ShapeDtypeStruct((BATCH, 128), jnp.int32)
                   for _ in range(n_out))

    @functools.partial(
        pl.kernel,
        out_type=out_ty,
        mesh=mesh,
        scratch_types=[
            pltpu.VMEM((B_PER_W,), jnp.int32),
            pltpu.VMEM((B_PER_W, 128), jnp.int32),
            pltpu.SemaphoreType.DMA,
        ],
        compiler_params=pltpu.CompilerParams(use_tc_tiling_on_sc=True),
    )
    def sck(*refs):
        args = refs[:2 * n_out]
        outs = refs[2 * n_out:3 * n_out]
        idx_v, rows_v, sem = refs[3 * n_out:]
        wid = lax.axis_index("s") * NUM_CORES + lax.axis_index("c")
        base = wid * B_PER_W
        for k in range(n_out):
            tab, idx_hbm, out_hbm = args[2 * k], args[2 * k + 1], outs[k]
            pltpu.sync_copy(idx_hbm.at[pl.ds(base, B_PER_W)], idx_v)
            pltpu.async_copy(tab.at[idx_v], rows_v, sem).wait()
            pltpu.sync_copy(rows_v, out_hbm.at[pl.ds(base, B_PER_W)])

    flat = []
    for tab, idx in tabs_and_idx:
        flat += [tab, idx]
    return sck(*flat)


_CB = 2048


def _score_body(hl_ref, hh_ref, rl_ref, tl_ref, th_ref, bits_ref, o_ref):
    bits = bits_ref[...]
    bits64 = bits[:, :EMBED_DIM]

    def unpack(lo_ref, hi_ref, k):
        if hi_ref is None:
            x = lo_ref[...]
        else:
            use_lo = ((bits >> (k + 2)) & 1) == 1
            x = jnp.where(use_lo, lo_ref[...], hi_ref[...])
        lane_hi = ((bits64 >> k) & 1) == 1
        v = jnp.where(lane_hi, x[:, EMBED_DIM:], x[:, :EMBED_DIM])
        word_hi = ((bits64 >> (k + 1)) & 1) == 1
        patt = jnp.where(word_hi, v & jnp.int32(-65536), v << 16)
        return jax.lax.bitcast_convert_type(patt, jnp.float32)

    hv = unpack(hl_ref, hh_ref, 0)
    rv = unpack(rl_ref, None, 4)
    tv = unpack(tl_ref, th_ref, 7)
    score = jnp.sum(hv * rv * tv, axis=1)
    o_ref[...] = jax.nn.sigmoid(score)


def _tc_score(hl, hh, rl, tl, th, bits):
    g = BATCH // _CB
    spec = pl.BlockSpec((_CB, 128), lambda i: (i, 0))
    bits_b = jnp.broadcast_to(bits[:, None], (BATCH, 128))
    out = pl.pallas_call(
        _score_body,
        grid=(g,),
        in_specs=[spec] * 6,
        out_specs=pl.BlockSpec((_CB,), lambda i: (i,)),
        out_shape=jax.ShapeDtypeStruct((BATCH,), jnp.float32),
    )(hl, hh, rl, tl, th, bits_b)
    return out


def _rowid(g):
    return ((g >> _LW) << (_LW - 2)) | (g & (_Q - 1))


def _selbits(g):
    return ((g >> (_LW - 2)) & 1) | (((g >> (_LW - 1)) & 1) << 1)


def kernel(h, r, t, entity_table, relation_table):
    et = entity_table.T
    we_lo = _transpose_pack(et, _NB_LO, 0)
    wr = _transpose_pack(relation_table.T, 1, 0)
    we_hi = _transpose_pack(et, _NBLK - _NB_LO, _NB_LO)

    h_lo, t_lo = h < _S, t < _S
    hj_lo = jnp.where(h_lo, _rowid(h), 0)
    hj_hi = jnp.where(h_lo, 0, _rowid(h) - _ROWS_LO)
    tj_lo = jnp.where(t_lo, _rowid(t), 0)
    tj_hi = jnp.where(t_lo, 0, _rowid(t) - _ROWS_LO)

    hl, rl, tl = _sc_gather(
        [(we_lo, hj_lo), (wr, _rowid(r)), (we_lo, tj_lo)], 3)
    hh, th = _sc_gather([(we_hi, hj_hi), (we_hi, tj_hi)], 2)

    bits = (_selbits(h)
            | (h_lo.astype(jnp.int32) << 2)
            | (_selbits(r) << 4)
            | (_selbits(t) << 7)
            | (t_lo.astype(jnp.int32) << 9))
    return _tc_score(hl, hh, rl, tl, th, bits)


# speedup vs baseline: 6.6796x; 6.6796x over previous
import functools

import jax
import jax.numpy as jnp
from jax import lax
from jax.experimental import pallas as pl
from jax.experimental.pallas import tpu as pltpu
from jax.experimental.pallas import tpu_sc as plsc

NUM_ENTITIES = 1000000
NUM_RELATIONS = 1000
EMBED_DIM = 64
BATCH = 16384

NUM_CORES = 2
NUM_SUBCORES = 16
NUM_WORKERS = NUM_CORES * NUM_SUBCORES
B_PER_W = BATCH // NUM_WORKERS

_W = 32768
_Q = _W // 4
_LW = _W.bit_length() - 1

_NBLK = (NUM_ENTITIES + _W - 1) // _W
_NB_LO = 16
_S = _NB_LO * _W
_ROWS_LO = _NB_LO * _Q


def _tr_body(x_ref, o_ref):
    x = x_ref[...].astype(jnp.bfloat16)
    y = jnp.transpose(x)
    u = jax.lax.bitcast_convert_type(y, jnp.uint16).astype(jnp.int32)
    a, b, c, d = (u[0:_Q], u[_Q:2 * _Q], u[2 * _Q:3 * _Q], u[3 * _Q:4 * _Q])
    p1 = a | (c << 16)
    p2 = b | (d << 16)
    o_ref[...] = jnp.concatenate([p1, p2], axis=1)


def _transpose_pack(et, nblk, blk0):
    return pl.pallas_call(
        _tr_body,
        grid=(nblk,),
        in_specs=[pl.BlockSpec((64, _W), lambda i: (0, i + blk0))],
        out_specs=pl.BlockSpec((_Q, 128), lambda i: (i, 0)),
        out_shape=jax.ShapeDtypeStruct((nblk * _Q, 128), jnp.int32),
    )(et)


def _sc_gather(tabs_and_idx, n_out):
    mesh = plsc.VectorSubcoreMesh(
        core_axis_name="c", subcore_axis_name="s",
        num_cores=NUM_CORES, num_subcores=NUM_SUBCORES)
    out_ty = tuple(jax.ShapeDtypeStruct((BATCH, 128), jnp.int32)
                   for _ in range(n_out))

    @functools.partial(
        pl.kernel,
        out_type=out_ty,
        mesh=mesh,
        scratch_types=[
            pltpu.VMEM((B_PER_W,), jnp.int32),
            pltpu.VMEM((B_PER_W, 128), jnp.int32),
            pltpu.SemaphoreType.DMA,
        ],
        compiler_params=pltpu.CompilerParams(use_tc_tiling_on_sc=True),
    )
    def sck(*refs):
        args = refs[:2 * n_out]
        outs = refs[2 * n_out:3 * n_out]
        idx_v, rows_v, sem = refs[3 * n_out:]
        wid = lax.axis_index("s") * NUM_CORES + lax.axis_index("c")
        base = wid * B_PER_W
        for k in range(n_out):
            tab, idx_hbm, out_hbm = args[2 * k], args[2 * k + 1], outs[k]
            pltpu.sync_copy(idx_hbm.at[pl.ds(base, B_PER_W)], idx_v)
            pltpu.async_copy(tab.at[idx_v], rows_v, sem).wait()
            pltpu.sync_copy(rows_v, out_hbm.at[pl.ds(base, B_PER_W)])

    flat = []
    for tab, idx in tabs_and_idx:
        flat += [tab, idx]
    return sck(*flat)


_CB = 2048


def _score_body(hl_ref, hh_ref, rl_ref, tl_ref, th_ref, bits_ref, o_ref):
    bits = bits_ref[...]
    bits64 = bits[:, :EMBED_DIM]

    def unpack(lo_ref, hi_ref, k):
        if hi_ref is None:
            x = lo_ref[...]
        else:
            use_lo = ((bits >> (k + 2)) & 1) == 1
            x = jnp.where(use_lo, lo_ref[...], hi_ref[...])
        lane_hi = ((bits64 >> k) & 1) == 1
        v = jnp.where(lane_hi, x[:, EMBED_DIM:], x[:, :EMBED_DIM])
        word_hi = ((bits64 >> (k + 1)) & 1) == 1
        patt = jnp.where(word_hi, v & jnp.int32(-65536), v << 16)
        return jax.lax.bitcast_convert_type(patt, jnp.float32)

    hv = unpack(hl_ref, hh_ref, 0)
    rv = unpack(rl_ref, None, 4)
    tv = unpack(tl_ref, th_ref, 7)
    score = jnp.sum(hv * rv * tv, axis=1)
    o_ref[...] = jax.nn.sigmoid(score)


def _tc_score(hl, hh, rl, tl, th, bits):
    g = BATCH // _CB
    spec = pl.BlockSpec((_CB, 128), lambda i: (i, 0))
    bits_b = jnp.broadcast_to(bits[:, None], (BATCH, 128))
    out = pl.pallas_call(
        _score_body,
        grid=(g,),
        in_specs=[spec] * 6,
        out_specs=pl.BlockSpec((_CB,), lambda i: (i,)),
        out_shape=jax.ShapeDtypeStruct((BATCH,), jnp.float32),
    )(hl, hh, rl, tl, th, bits_b)
    return out


def _rowid(g):
    return ((g >> _LW) << (_LW - 2)) | (g & (_Q - 1))


def _selbits(g):
    return ((g >> (_LW - 2)) & 1) | (((g >> (_LW - 1)) & 1) << 1)


def kernel(h, r, t, entity_table, relation_table):
    et = entity_table.T
    we_lo = _transpose_pack(et, _NB_LO, 0)
    wr = _transpose_pack(relation_table.T, 1, 0)
    we_hi = _transpose_pack(et, _NBLK - _NB_LO, _NB_LO)

    h_lo, t_lo = h < _S, t < _S
    spread_h, spread_t = h & (_Q - 1), t & (_Q - 1)
    hj_lo = jnp.where(h_lo, _rowid(h), spread_h)
    hj_hi = jnp.where(h_lo, spread_h, _rowid(h) - _ROWS_LO)
    tj_lo = jnp.where(t_lo, _rowid(t), spread_t)
    tj_hi = jnp.where(t_lo, spread_t, _rowid(t) - _ROWS_LO)

    hl, rl, tl = _sc_gather(
        [(we_lo, hj_lo), (wr, _rowid(r)), (we_lo, tj_lo)], 3)
    hh, th = _sc_gather([(we_hi, hj_hi), (we_hi, tj_hi)], 2)

    bits = (_selbits(h)
            | (h_lo.astype(jnp.int32) << 2)
            | (_selbits(r) << 4)
            | (_selbits(t) << 7)
            | (t_lo.astype(jnp.int32) << 9))
    return _tc_score(hl, hh, rl, tl, th, bits)


# speedup vs baseline: 7.1437x; 1.0695x over previous
import functools

import jax
import jax.numpy as jnp
from jax import lax
from jax.experimental import pallas as pl
from jax.experimental.pallas import tpu as pltpu
from jax.experimental.pallas import tpu_sc as plsc

NUM_ENTITIES = 1000000
NUM_RELATIONS = 1000
EMBED_DIM = 64
BATCH = 16384

NUM_CORES = 2
NUM_SUBCORES = 16
NUM_WORKERS = NUM_CORES * NUM_SUBCORES
B_PER_W = BATCH // NUM_WORKERS

_W = 32768
_Q = _W // 4


def _tr_body(x_ref, o_ref):
    x = x_ref[...].astype(jnp.bfloat16)
    y = jnp.transpose(x)
    u = jax.lax.bitcast_convert_type(y, jnp.uint16).astype(jnp.int32)
    a, b, c, d = (u[0:_Q], u[_Q:2 * _Q], u[2 * _Q:3 * _Q], u[3 * _Q:4 * _Q])
    p1 = a | (c << 16)
    p2 = b | (d << 16)
    o_ref[...] = jnp.concatenate([p1, p2], axis=1)


def _transpose_pack(et, n):
    nblk = (n + _W - 1) // _W
    return pl.pallas_call(
        _tr_body,
        grid=(nblk,),
        in_specs=[pl.BlockSpec((64, _W), lambda i: (0, i))],
        out_specs=pl.BlockSpec((_Q, 128), lambda i: (i, 0)),
        out_shape=jax.ShapeDtypeStruct((nblk * _Q, 128), jnp.int32),
    )(et)


def _sc_gather(we, wr, hj, rj, tj):
    mesh = plsc.VectorSubcoreMesh(
        core_axis_name="c", subcore_axis_name="s",
        num_cores=NUM_CORES, num_subcores=NUM_SUBCORES)
    out_ty = jax.ShapeDtypeStruct((BATCH, 128), jnp.int32)

    @functools.partial(
        pl.kernel,
        out_type=(out_ty, out_ty, out_ty),
        mesh=mesh,
        scratch_types=[
            pltpu.VMEM((B_PER_W,), jnp.int32),
            pltpu.VMEM((B_PER_W, 128), jnp.int32),
            pltpu.SemaphoreType.DMA,
        ],
        compiler_params=pltpu.CompilerParams(use_tc_tiling_on_sc=True),
    )
    def sck(we_hbm, wr_hbm, hj_hbm, rj_hbm, tj_hbm,
            hw_hbm, rw_hbm, tw_hbm, idx_v, rows_v, sem):
        wid = lax.axis_index("s") * NUM_CORES + lax.axis_index("c")
        base = wid * B_PER_W
        for tab, idx_hbm, out_hbm in (
                (we_hbm, hj_hbm, hw_hbm),
                (wr_hbm, rj_hbm, rw_hbm),
                (we_hbm, tj_hbm, tw_hbm)):
            pltpu.sync_copy(idx_hbm.at[pl.ds(base, B_PER_W)], idx_v)
            pltpu.async_copy(tab.at[idx_v], rows_v, sem).wait()
            pltpu.sync_copy(rows_v, out_hbm.at[pl.ds(base, B_PER_W)])

    return sck(we, wr, hj, rj, tj)


_CB = 2048


def _score_body(hw_ref, rw_ref, tw_ref, bits_ref, o_ref):
    bits = bits_ref[...][:, :EMBED_DIM]

    def unpack(x_ref, k):
        x = x_ref[...]
        lane_hi = ((bits >> k) & 1) == 1
        v = jnp.where(lane_hi, x[:, EMBED_DIM:], x[:, :EMBED_DIM])
        word_hi = ((bits >> (k + 1)) & 1) == 1
        patt = jnp.where(word_hi, v & jnp.int32(-65536), v << 16)
        return jax.lax.bitcast_convert_type(patt, jnp.float32)

    hv = unpack(hw_ref, 0)
    rv = unpack(rw_ref, 2)
    tv = unpack(tw_ref, 4)
    score = jnp.sum(hv * rv * tv, axis=1)
    o_ref[...] = jax.nn.sigmoid(score)


def _tc_score(hw, rw, tw, bits):
    g = BATCH // _CB
    spec = pl.BlockSpec((_CB, 128), lambda i: (i, 0))
    bits_b = jnp.broadcast_to(bits[:, None], (BATCH, 128))
    out = pl.pallas_call(
        _score_body,
        grid=(g,),
        in_specs=[spec, spec, spec, spec],
        out_specs=pl.BlockSpec((_CB,), lambda i: (i,)),
        out_shape=jax.ShapeDtypeStruct((BATCH,), jnp.float32),
    )(hw, rw, tw, bits_b)
    return out


_LW = _W.bit_length() - 1


def _rowid(g):
    return ((g >> _LW) << (_LW - 2)) | (g & (_Q - 1))


def _selbits(g):
    return ((g >> (_LW - 2)) & 1) | (((g >> (_LW - 1)) & 1) << 1)


def kernel(h, r, t, entity_table, relation_table):
    we = _transpose_pack(entity_table.T, NUM_ENTITIES)
    wr = _transpose_pack(relation_table.T, NUM_RELATIONS)
    hw, rw, tw = _sc_gather(we, wr, _rowid(h), _rowid(r), _rowid(t))
    bits = _selbits(h) | (_selbits(r) << 2) | (_selbits(t) << 4)
    return _tc_score(hw, rw, tw, bits)
